# Initial kernel scaffold; baseline (speedup 1.0000x reference)
import jax
import jax.numpy as jnp
from jax.experimental import pallas as pl


def kernel(input_ids, weight, scales):
    raise NotImplementedError("write your pallas kernel here")


# speedup vs baseline: 1.3262x; 1.3262x over previous
import functools

import jax
import jax.numpy as jnp
from jax import lax
from jax.experimental import pallas as pl
from jax.experimental.pallas import tpu as pltpu
from jax.experimental.pallas import tpu_sc as plsc

D = 64
LANES = 16
NC = 2
NS = 16
NW = NC * NS

S = 128

SIGN_MASK = -2147483648


@functools.lru_cache(maxsize=None)
def _make_kernel(N):
    assert N % (NW * S) == 0
    per_w = N // NW
    n_chunks = per_w // S
    mesh = plsc.VectorSubcoreMesh(core_axis_name="c", subcore_axis_name="s")

    @functools.partial(
        pl.kernel,
        mesh=mesh,
        compiler_params=pltpu.CompilerParams(use_tc_tiling_on_sc=False),
        out_type=jax.ShapeDtypeStruct((N, D), jnp.float32),
        scratch_types=[
            pltpu.VMEM((n_chunks, S), jnp.int32),
            pltpu.VMEM((S, D), jnp.float32),
            pltpu.VMEM((S,), jnp.float32),
            pltpu.SemaphoreType.DMA,
        ],
    )
    def k(ids_hbm, w_hbm, sc_hbm, out_hbm, idx_v, rows_v, scf_v, sem):
        wid = lax.axis_index("s") * NC + lax.axis_index("c")
        base = wid * per_w

        pltpu.sync_copy(ids_hbm.at[wid], idx_v)

        def chunk_body(c, carry):
            idx_c = idx_v.at[c]

            pltpu.async_copy(sc_hbm.at[idx_c], scf_v, sem).wait()
            pltpu.async_copy(w_hbm.at[idx_c], rows_v, sem).wait()

            def group_body(g, carry2):
                sg = jnp.maximum(scf_v[pl.ds(g * LANES, LANES)],
                                 jnp.float32(1e-8))
                nsg = -sg
                for kk in range(LANES):
                    splat = jnp.broadcast_to(sg[kk], (LANES,))
                    nsplat = jnp.broadcast_to(nsg[kk], (LANES,))
                    r = g * LANES + kk
                    for j in range(D // LANES):
                        w = rows_v[r, pl.ds(j * LANES, LANES)]
                        rows_v[r, pl.ds(j * LANES, LANES)] = jnp.where(
                            w < 0, nsplat, splat)
                return carry2

            lax.fori_loop(0, S // LANES, group_body, 0)

            pltpu.sync_copy(rows_v, out_hbm.at[pl.ds(base + c * S, S)])
            return carry

        lax.fori_loop(0, n_chunks, chunk_body, 0)

    return k


def kernel(input_ids, weight, scales):
    B, L = input_ids.shape
    N = B * L
    ids = input_ids.reshape(NW, N // (NW * S), S).astype(jnp.int32)
    sc_flat = scales.reshape(-1)
    out = _make_kernel(N)(ids, weight, sc_flat)
    return out.reshape(B, L, D)
